# baseline (speedup 1.0000x reference)
import functools

import jax
import jax.numpy as jnp
from jax import lax
from jax.experimental import pallas as pl
from jax.experimental.pallas import tpu as pltpu
from jax.experimental.pallas import tpu_sc as plsc

_NUM_FIELDS = 26
_FIELD_DIM = 100000
_EMBED_DIM = 16
_BATCH = 4096
_TABLE_ROWS = _NUM_FIELDS * _FIELD_DIM

_NC, _NS, _L = 2, 16, 16
_NW = _NC * _NS
_PAIRS = _NUM_FIELDS * _EMBED_DIM
_PPW = _PAIRS // _NW
_CHUNK = 128
_NCHUNK = _BATCH // _CHUNK

_mesh = plsc.VectorSubcoreMesh(
    core_axis_name="c", subcore_axis_name="s", num_cores=_NC, num_subcores=_NS
)


@functools.partial(
    pl.kernel,
    out_type=jax.ShapeDtypeStruct((_NUM_FIELDS * _EMBED_DIM * _BATCH,), jnp.float32),
    mesh=_mesh,
    scratch_types=[
        pltpu.VMEM((_PPW * _BATCH,), jnp.int32),
        pltpu.VMEM((_PPW * _BATCH,), jnp.float32),
        pltpu.SemaphoreType.DMA,
    ],
    compiler_params=pltpu.CompilerParams(use_tc_tiling_on_sc=False),
)
def _embed_gather(xt_hbm, table_hbm, out_hbm, idx_v, rows_v, gsem):
    wid = lax.axis_index("s") * _NC + lax.axis_index("c")
    p0 = wid * _PPW

    for k in range(_PPW):
        f = (p0 + k) // _EMBED_DIM
        pltpu.sync_copy(xt_hbm.at[pl.ds(f * _BATCH, _BATCH)], idx_v.at[pl.ds(k * _BATCH, _BATCH)])

    for k in range(_PPW):
        p = p0 + k
        f = p // _EMBED_DIM
        d = lax.rem(p, _EMBED_DIM)
        base = d * _TABLE_ROWS + f * _FIELD_DIM
        base = pl.multiple_of(base, 8)
        for j in range(_NCHUNK):
            pltpu.make_async_copy(
                table_hbm.at[pl.ds(base, _FIELD_DIM)].at[
                    idx_v.at[pl.ds(k * _BATCH + j * _CHUNK, _CHUNK)]
                ],
                rows_v.at[pl.ds(k * _BATCH + j * _CHUNK, _CHUNK)],
                gsem,
            ).start()

    pltpu.make_async_copy(
        table_hbm.at[pl.ds(0, _PPW * _BATCH)], rows_v, gsem
    ).wait()
    pltpu.sync_copy(
        rows_v, out_hbm.at[pl.ds(p0 * _BATCH, _PPW * _BATCH)]
    )


def kernel(x, table):
    xt_flat = jnp.swapaxes(x, 0, 1).reshape(_NUM_FIELDS * _BATCH)
    tt_flat = jnp.swapaxes(table, 0, 1).reshape(_EMBED_DIM * _TABLE_ROWS)
    out = _embed_gather(xt_flat, tt_flat)
    out = out.reshape(_NUM_FIELDS, _EMBED_DIM, _BATCH)
    return jnp.transpose(out, (2, 0, 1))
